# Initial kernel scaffold; baseline (speedup 1.0000x reference)
import jax
import jax.numpy as jnp
from jax.experimental import pallas as pl


def kernel(h, edge_index, params):
    raise NotImplementedError("write your pallas kernel here")


# speedup vs baseline: 20.4485x; 20.4485x over previous
import functools

import jax
import jax.numpy as jnp
from jax import lax
from jax.experimental import pallas as pl
from jax.experimental.pallas import tpu as pltpu
from jax.experimental.pallas import tpu_sc as plsc

NDIM = 128
N_NODES = 10000
E = 320000
N_SUBCORES = 16
EDGES_PER_TILE = E // N_SUBCORES
CHUNK = 80
NCHUNK = EDGES_PER_TILE // CHUNK
ACC_ROWS = 10240
ROWS_PER_TILE = ACC_ROWS // N_SUBCORES
ZCH = 80
IBLK = 50
NBLK = NCHUNK // IBLK


def _sc_deg_body(sidx, zeros, ones, out, idxs_v, ones_v, zbuf, acc):
    c = lax.axis_index("c")
    s = lax.axis_index("s")
    pltpu.sync_copy(ones, ones_v)
    pltpu.sync_copy(zeros, zbuf)
    for k in range(ROWS_PER_TILE // ZCH):
        pltpu.sync_copy(zbuf, acc.at[pl.ds(s * ROWS_PER_TILE + k * ZCH, ZCH)])
    plsc.subcore_barrier()

    def blk(b, _):
        pltpu.sync_copy(sidx.at[c, s, b], idxs_v)

        def body(j, _):
            pltpu.sync_copy(ones_v, acc.at[idxs_v.at[j]], add=True)
            return _

        return lax.fori_loop(0, IBLK, body, _)

    lax.fori_loop(0, NBLK, blk, None)
    plsc.subcore_barrier()
    for k in range(ROWS_PER_TILE // ZCH):
        r0 = s * ROWS_PER_TILE + k * ZCH
        pltpu.sync_copy(acc.at[pl.ds(r0, ZCH)], zbuf)
        pltpu.sync_copy(zbuf, out.at[c, pl.ds(r0, ZCH)])


def _sc_degrees(sidx, zeros, ones):
    mesh = plsc.VectorSubcoreMesh(core_axis_name="c", subcore_axis_name="s")
    kfn = functools.partial(
        pl.kernel,
        mesh=mesh,
        out_type=jax.ShapeDtypeStruct((2, ACC_ROWS, NDIM), jnp.float32),
        scratch_types=[
            pltpu.VMEM((IBLK, CHUNK), jnp.int32),
            pltpu.VMEM((CHUNK, NDIM), jnp.float32),
            pltpu.VMEM((ZCH, NDIM), jnp.float32),
            pltpu.VMEM_SHARED((ACC_ROWS, NDIM), jnp.float32),
        ],
    )(_sc_deg_body)
    return kfn(sidx, zeros, ones)


def _sc_spmm_body(htab, gidx, sidx, zeros, out, idxg_v, idxs_v, rows_a, rows_b,
                  acc, sem_a, sem_b):
    c = lax.axis_index("c")
    s = lax.axis_index("s")

    pltpu.sync_copy(zeros, rows_a)
    for k in range(ROWS_PER_TILE // ZCH):
        pltpu.sync_copy(rows_a, acc.at[pl.ds(s * ROWS_PER_TILE + k * ZCH, ZCH)])
    plsc.subcore_barrier()

    def blk(b, _):
        pltpu.sync_copy(gidx.at[c, s, b], idxg_v)
        pltpu.sync_copy(sidx.at[c, s, b], idxs_v)
        pltpu.async_copy(htab.at[idxg_v.at[0]], rows_a, sem_a)

        def body(i, carry):
            j = 2 * i
            cp_b = pltpu.async_copy(htab.at[idxg_v.at[j + 1]], rows_b, sem_b)
            pltpu.make_async_copy(htab.at[idxg_v.at[j]], rows_a, sem_a).wait()
            pltpu.sync_copy(rows_a, acc.at[idxs_v.at[j]], add=True)
            pltpu.async_copy(htab.at[idxg_v.at[j + 2]], rows_a, sem_a)
            cp_b.wait()
            pltpu.sync_copy(rows_b, acc.at[idxs_v.at[j + 1]], add=True)
            return carry

        lax.fori_loop(0, (IBLK - 2) // 2, body, 0)
        cp_b = pltpu.async_copy(htab.at[idxg_v.at[IBLK - 1]], rows_b, sem_b)
        pltpu.make_async_copy(htab.at[idxg_v.at[IBLK - 2]], rows_a, sem_a).wait()
        pltpu.sync_copy(rows_a, acc.at[idxs_v.at[IBLK - 2]], add=True)
        cp_b.wait()
        pltpu.sync_copy(rows_b, acc.at[idxs_v.at[IBLK - 1]], add=True)
        return _

    lax.fori_loop(0, NBLK, blk, None)

    plsc.subcore_barrier()
    for k in range(ROWS_PER_TILE // ZCH):
        r0 = s * ROWS_PER_TILE + k * ZCH
        pltpu.sync_copy(acc.at[pl.ds(r0, ZCH)], rows_a)
        pltpu.sync_copy(rows_a, out.at[c, pl.ds(r0, ZCH)])


def _sc_spmm(htab, gidx, sidx, zeros):
    mesh = plsc.VectorSubcoreMesh(core_axis_name="c", subcore_axis_name="s")
    kfn = functools.partial(
        pl.kernel,
        mesh=mesh,
        out_type=jax.ShapeDtypeStruct((2, ACC_ROWS, NDIM), jnp.float32),
        scratch_types=[
            pltpu.VMEM((IBLK, CHUNK), jnp.int32),
            pltpu.VMEM((IBLK, CHUNK), jnp.int32),
            pltpu.VMEM((CHUNK, NDIM), jnp.float32),
            pltpu.VMEM((CHUNK, NDIM), jnp.float32),
            pltpu.VMEM_SHARED((ACC_ROWS, NDIM), jnp.float32),
            pltpu.SemaphoreType.DMA,
            pltpu.SemaphoreType.DMA,
        ],
    )(_sc_spmm_body)
    return kfn(htab, gidx, sidx, zeros)


def _bfr(x):
    return lax.reduce_precision(x, exponent_bits=8, mantissa_bits=7)


def _bfr_k(x):
    return x.astype(jnp.bfloat16).astype(jnp.float32)


def _gru_body(h_ref, sf_ref, sr_ref, df_ref, dr_ref, af_ref, ar_ref, bf_ref,
              br_ref, wih_ref, whh_ref, mbf_ref, mbr_ref, bih_ref, bhh_ref,
              out_ref):
    HI = lax.Precision.HIGHEST
    h = h_ref[...]
    hb = _bfr_k(h)
    df = df_ref[...]
    dr = dr_ref[...]
    f32 = jnp.float32
    aggr = (jnp.dot(sf_ref[...], af_ref[...], preferred_element_type=f32, precision=HI)
            + jnp.dot(sr_ref[...], ar_ref[...], preferred_element_type=f32, precision=HI)
            + df * (jnp.dot(hb, bf_ref[...], preferred_element_type=f32, precision=HI)
                    + mbf_ref[...])
            + dr * (jnp.dot(hb, br_ref[...], preferred_element_type=f32, precision=HI)
                    + mbr_ref[...]))
    gi = jnp.dot(_bfr_k(aggr), wih_ref[...], preferred_element_type=f32,
                 precision=HI) + bih_ref[...]
    gh = jnp.dot(hb, whh_ref[...], preferred_element_type=f32,
                 precision=HI) + bhh_ref[...]
    r = jax.nn.sigmoid(gi[:, :NDIM] + gh[:, :NDIM])
    z = jax.nn.sigmoid(gi[:, NDIM:2 * NDIM] + gh[:, NDIM:2 * NDIM])
    nn_ = jnp.tanh(gi[:, 2 * NDIM:] + r * gh[:, 2 * NDIM:])
    out_ref[...] = (1.0 - z) * nn_ + z * h


def _gru_update(hh, sf, sr, df, dr, Af, Ar, Bf, Br, Wih, Whh, mbf, mbr, bih, bhh):
    BM = 1000
    grid = (N_NODES // BM,)
    full = lambda shape: pl.BlockSpec(shape, lambda i: (0, 0))
    return pl.pallas_call(
        _gru_body,
        grid=grid,
        in_specs=[
            pl.BlockSpec((BM, NDIM), lambda i: (i, 0)),
            pl.BlockSpec((BM, NDIM), lambda i: (i, 0)),
            pl.BlockSpec((BM, NDIM), lambda i: (i, 0)),
            pl.BlockSpec((BM, 1), lambda i: (i, 0)),
            pl.BlockSpec((BM, 1), lambda i: (i, 0)),
            full((NDIM, 2 * NDIM)), full((NDIM, 2 * NDIM)),
            full((NDIM, 2 * NDIM)), full((NDIM, 2 * NDIM)),
            full((2 * NDIM, 3 * NDIM)), full((NDIM, 3 * NDIM)),
            full((1, 2 * NDIM)), full((1, 2 * NDIM)),
            full((1, 3 * NDIM)), full((1, 3 * NDIM)),
        ],
        out_specs=pl.BlockSpec((BM, NDIM), lambda i: (i, 0)),
        out_shape=jax.ShapeDtypeStruct((N_NODES, NDIM), jnp.float32),
    )(hh, sf, sr, df, dr, Af, Ar, Bf, Br, Wih, Whh, mbf, mbr, bih, bhh)


def _final_body(h_ref, fmw_ref, fmb_ref, fmiw_ref, fmib_ref,
                hn_ref, hg_ref, hgi_ref, idx_count):
    HI = lax.Precision.HIGHEST
    pid = pl.program_id(0)
    h = h_ref[...]
    nrm = jnp.sqrt(jnp.sum(h * h, axis=1, keepdims=True))
    hn = h / jnp.maximum(nrm, 1e-12)
    hn_ref[...] = hn
    mbar = jnp.sum(_bfr_k(hn), axis=0, keepdims=True) * (1.0 / idx_count)
    g = jnp.dot(mbar, fmw_ref[...], preferred_element_type=jnp.float32,
                precision=HI) + fmb_ref[...]
    gn = jnp.sqrt(jnp.sum(g * g, axis=1, keepdims=True))
    hg_ref[pl.ds(pid, 1), :] = g / jnp.maximum(gn, 1e-12)
    gi = jnp.dot(mbar, fmiw_ref[...], preferred_element_type=jnp.float32,
                 precision=HI) + fmib_ref[...]
    gin = jnp.sqrt(jnp.sum(gi * gi, axis=1, keepdims=True))
    hgi_ref[pl.ds(pid, 1), :] = gi / jnp.maximum(gin, 1e-12)


def _final_stage(hh, fmw, fmb, fmiw, fmib, nbatch, idx_count):
    full = lambda shape: pl.BlockSpec(shape, lambda i: (0, 0))
    return pl.pallas_call(
        functools.partial(_final_body, idx_count=float(idx_count)),
        grid=(nbatch,),
        in_specs=[
            pl.BlockSpec((idx_count, NDIM), lambda i: (i, 0)),
            full((NDIM, NDIM)), full((1, NDIM)),
            full((NDIM, NDIM)), full((1, NDIM)),
        ],
        out_specs=[
            pl.BlockSpec((idx_count, NDIM), lambda i: (i, 0)),
            pl.BlockSpec((nbatch, NDIM), lambda i: (0, 0)),
            pl.BlockSpec((nbatch, NDIM), lambda i: (0, 0)),
        ],
        out_shape=[
            jax.ShapeDtypeStruct((nbatch * idx_count, NDIM), jnp.float32),
            jax.ShapeDtypeStruct((nbatch, NDIM), jnp.float32),
            jax.ShapeDtypeStruct((nbatch, NDIM), jnp.float32),
        ],
    )(hh, fmw, fmb, fmiw, fmib)


def kernel(h, edge_index, params):
    nbatch, idx, _ = h.shape
    hh = h.reshape(-1, NDIM)
    src = edge_index[0]
    dst = edge_index[1]
    gidx = jnp.stack([src, dst]).reshape(2, N_SUBCORES, NBLK, IBLK, CHUNK)
    sidx = jnp.stack([dst, src]).reshape(2, N_SUBCORES, NBLK, IBLK, CHUNK)
    zeros = jnp.zeros((ZCH, NDIM), jnp.float32)
    ones = jnp.ones((CHUNK, NDIM), jnp.float32)
    assert ZCH == CHUNK

    deg = _sc_degrees(sidx, zeros, ones)
    df = deg[0, :N_NODES, :1]
    dr = deg[1, :N_NODES, :1]

    for lp in params["layers"]:
        Af = _bfr(lp["msg_W"].T[:NDIM])
        Bf = _bfr(lp["msg_W"].T[NDIM:])
        Ar = _bfr(lp["msg_rev_W"].T[:NDIM])
        Br = _bfr(lp["msg_rev_W"].T[NDIM:])
        Wih = _bfr(lp["W_ih"].T)
        Whh = _bfr(lp["W_hh"].T)
        mbf = lp["msg_b"][None, :]
        mbr = lp["msg_rev_b"][None, :]
        bih = lp["b_ih"][None, :]
        bhh = lp["b_hh"][None, :]
        s_full = _sc_spmm(_bfr(hh), gidx, sidx, zeros)
        hh = _gru_update(hh, s_full[0, :N_NODES], s_full[1, :N_NODES], df, dr,
                         Af, Ar, Bf, Br, Wih, Whh, mbf, mbr, bih, bhh)

    hn, h_G, h_G_init = _final_stage(
        hh, _bfr(params["fm_W"].T), params["fm_b"][None, :],
        _bfr(params["fmi_W"].T), params["fmi_b"][None, :], nbatch, idx)
    return (hn.reshape(nbatch, idx, NDIM), h_G, h_G_init)
